# baseline (speedup 1.0000x reference)
import functools

import jax
import jax.numpy as jnp
from jax import lax
from jax.experimental import pallas as pl
from jax.experimental.pallas import tpu as pltpu
from jax.experimental.pallas import tpu_sc as plsc

N_EMBEDDINGS = 8192
EMBEDDING_DIM = 64
BETA = 0.25

M_BLK = 1024
C_BLK = 2048
N_CHUNKS = N_EMBEDDINGS // C_BLK
KPACK = 4


def _rne_bf16(x):
    u = jax.lax.bitcast_convert_type(x, jnp.uint32)
    r = (u + jnp.uint32(0x7FFF) + ((u >> 16) & jnp.uint32(1))) \
        & jnp.uint32(0xFFFF0000)
    return jax.lax.bitcast_convert_type(r, jnp.float32)


def _argmin_kernel(z_ref, bm2_ref, s1_ref, s2_ref, idx_ref):
    z = z_ref[...]
    s1 = s1_ref[...]
    z4 = jnp.concatenate([z] * KPACK, axis=1)

    run_min = jnp.full((M_BLK,), jnp.inf, dtype=jnp.float32)
    col_iota = jax.lax.broadcasted_iota(jnp.int32, (M_BLK, C_BLK), 1)

    def chunk_dot(k):
        bm2 = bm2_ref[:, pl.ds(k * C_BLK, C_BLK)]
        return jnp.dot(z4, bm2, preferred_element_type=jnp.float32)

    args = []
    mm2 = chunk_dot(0)
    for k in range(N_CHUNKS):
        mm2_next = chunk_dot(k + 1) if k + 1 < N_CHUNKS else None
        s2 = s2_ref[0, pl.ds(k * C_BLK, C_BLK)]
        d = (s1 + s2[None, :]) + mm2
        m_k = jnp.min(d, axis=1)
        a_k = jnp.min(
            jnp.where(d == m_k[:, None], col_iota, N_EMBEDDINGS), axis=1)
        better = m_k < run_min
        run_min = jnp.where(better, _rne_bf16(m_k), run_min)
        args.append((better, a_k))
        mm2 = mm2_next

    win_arg = jnp.zeros((M_BLK,), dtype=jnp.int32)
    for k, (better, a_k) in enumerate(args):
        win_arg = jnp.where(better, a_k + k * C_BLK, win_arg)
    idx_ref[...] = win_arg[:, None]


def _epilogue_kernel(z_ref, zq_ref, t_ref, loss_ref):
    i = pl.program_id(0)
    t = zq_ref[:, 0:EMBEDDING_DIM] - z_ref[...]
    t_ref[...] = t

    @pl.when(i == 0)
    def _init():
        loss_ref[...] = jnp.zeros((1, 1), jnp.float32)

    loss_ref[...] += jnp.sum(t * t).reshape(1, 1)


GATHER_W = 128


def _make_sc_gather(n_rows):
    info = plsc.get_sparse_core_info()
    n_workers = info.num_cores * info.num_subcores
    b_per_w = n_rows // n_workers
    mesh = plsc.VectorSubcoreMesh(core_axis_name="c", subcore_axis_name="s")

    @functools.partial(
        pl.kernel, mesh=mesh,
        out_type=jax.ShapeDtypeStruct((n_rows, GATHER_W), jnp.float32),
        scratch_types=[
            pltpu.VMEM((b_per_w,), jnp.int32),
            pltpu.VMEM((b_per_w, GATHER_W), jnp.float32),
            pltpu.SemaphoreType.DMA,
        ],
    )
    def sc_gather(table_hbm, idx_hbm, out_hbm, idx_v, rows_v, sem):
        wid = lax.axis_index("s") * info.num_cores + lax.axis_index("c")
        base = wid * b_per_w
        pltpu.sync_copy(idx_hbm.at[pl.ds(base, b_per_w)], idx_v)
        pltpu.async_copy(table_hbm.at[idx_v], rows_v, sem).wait()
        pltpu.sync_copy(rows_v, out_hbm.at[pl.ds(base, b_per_w)])

    return sc_gather


@jax.jit
def kernel(z, codebook):
    z_flat = z.reshape(-1, EMBEDDING_DIM)
    n_rows = z_flat.shape[0]
    s1 = jnp.sum(z_flat ** 2, axis=1, keepdims=True)
    s2 = jnp.sum(codebook ** 2, axis=1).reshape(1, -1)
    cbm2_t = (-2.0 * codebook).T
    sel = (jnp.arange(N_EMBEDDINGS) % KPACK)[None, :] \
        == jnp.arange(KPACK)[:, None]
    bm2 = (sel[:, None, :] * cbm2_t[None]).reshape(
        KPACK * EMBEDDING_DIM, N_EMBEDDINGS)

    grid = (n_rows // M_BLK,)
    idx = pl.pallas_call(
        _argmin_kernel,
        grid=grid,
        in_specs=[
            pl.BlockSpec((M_BLK, EMBEDDING_DIM), lambda i: (i, 0)),
            pl.BlockSpec(bm2.shape, lambda i: (0, 0)),
            pl.BlockSpec((M_BLK, 1), lambda i: (i, 0)),
            pl.BlockSpec((1, N_EMBEDDINGS), lambda i: (0, 0)),
        ],
        out_specs=pl.BlockSpec((M_BLK, 1), lambda i: (i, 0)),
        out_shape=jax.ShapeDtypeStruct((n_rows, 1), jnp.int32),
    )(z_flat, bm2, s1, s2)

    table128 = jnp.concatenate(
        [codebook, jnp.zeros_like(codebook)], axis=1)
    z_q = _make_sc_gather(n_rows)(table128, idx.reshape(-1))

    t, loss_sum = pl.pallas_call(
        _epilogue_kernel,
        grid=grid,
        in_specs=[
            pl.BlockSpec((M_BLK, EMBEDDING_DIM), lambda i: (i, 0)),
            pl.BlockSpec((M_BLK, GATHER_W), lambda i: (i, 0)),
        ],
        out_specs=[
            pl.BlockSpec((M_BLK, EMBEDDING_DIM), lambda i: (i, 0)),
            pl.BlockSpec((1, 1), lambda i: (0, 0)),
        ],
        out_shape=[
            jax.ShapeDtypeStruct((n_rows, EMBEDDING_DIM), jnp.float32),
            jax.ShapeDtypeStruct((1, 1), jnp.float32),
        ],
    )(z_flat, z_q)

    mean_sq = loss_sum[0, 0] / (n_rows * EMBEDDING_DIM)
    embedding_loss = mean_sq + BETA * mean_sq
    z_q_out = z + t.reshape(z.shape)
    return z_q_out, embedding_loss


# speedup vs baseline: 1.0619x; 1.0619x over previous
import functools

import jax
import jax.numpy as jnp
from jax import lax
from jax.experimental import pallas as pl
from jax.experimental.pallas import tpu as pltpu
from jax.experimental.pallas import tpu_sc as plsc

N_EMBEDDINGS = 8192
EMBEDDING_DIM = 64
BETA = 0.25

M_BLK = 1024
C_BLK = 2048
N_CHUNKS = N_EMBEDDINGS // C_BLK
KPACK = 4


def _rne_bf16(x):
    u = jax.lax.bitcast_convert_type(x, jnp.uint32)
    r = (u + jnp.uint32(0x7FFF) + ((u >> 16) & jnp.uint32(1))) \
        & jnp.uint32(0xFFFF0000)
    return jax.lax.bitcast_convert_type(r, jnp.float32)


def _argmin_kernel(z_ref, cbm2_ref, s1_ref, s2_ref, idx_ref):
    z = z_ref[...]
    s1 = s1_ref[...]

    run_min = jnp.full((M_BLK,), jnp.inf, dtype=jnp.float32)
    col_iota = jax.lax.broadcasted_iota(jnp.int32, (M_BLK, C_BLK), 1)

    def chunk_dot(k):
        cbm2 = cbm2_ref[pl.ds(k * C_BLK, C_BLK), :]
        return jax.lax.dot_general(
            z, cbm2, (((1,), (1,)), ((), ())),
            preferred_element_type=jnp.float32)

    args = []
    mm2 = chunk_dot(0)
    for k in range(N_CHUNKS):
        mm2_next = chunk_dot(k + 1) if k + 1 < N_CHUNKS else None
        s2 = s2_ref[0, pl.ds(k * C_BLK, C_BLK)]
        d = (s1 + s2[None, :]) + mm2
        m_k = jnp.min(d, axis=1)
        a_k = jnp.min(
            jnp.where(d == m_k[:, None], col_iota, N_EMBEDDINGS), axis=1)
        better = m_k < run_min
        run_min = jnp.where(better, _rne_bf16(m_k), run_min)
        args.append((better, a_k))
        mm2 = mm2_next

    win_arg = jnp.zeros((M_BLK,), dtype=jnp.int32)
    for k, (better, a_k) in enumerate(args):
        win_arg = jnp.where(better, a_k + k * C_BLK, win_arg)
    idx_ref[...] = win_arg[:, None]


def _epilogue_kernel(z_ref, zq_ref, idx_ref, t_ref, loss_ref):
    i = pl.program_id(0)
    odd = (idx_ref[...][:, 0] & 1) == 1
    zq = jnp.where(odd[:, None],
                   zq_ref[:, EMBEDDING_DIM:2 * EMBEDDING_DIM],
                   zq_ref[:, 0:EMBEDDING_DIM])
    t = zq - z_ref[...]
    t_ref[...] = t

    @pl.when(i == 0)
    def _init():
        loss_ref[...] = jnp.zeros((1, 1), jnp.float32)

    loss_ref[...] += jnp.sum(t * t).reshape(1, 1)


GATHER_W = 128


def _make_sc_gather(n_rows):
    info = plsc.get_sparse_core_info()
    n_workers = info.num_cores * info.num_subcores
    b_per_w = n_rows // n_workers
    mesh = plsc.VectorSubcoreMesh(core_axis_name="c", subcore_axis_name="s")

    @functools.partial(
        pl.kernel, mesh=mesh,
        out_type=jax.ShapeDtypeStruct((n_rows, GATHER_W), jnp.float32),
        scratch_types=[
            pltpu.VMEM((b_per_w,), jnp.int32),
            pltpu.VMEM((b_per_w, GATHER_W), jnp.float32),
            pltpu.SemaphoreType.DMA,
        ],
    )
    def sc_gather(table_hbm, idx_hbm, out_hbm, idx_v, rows_v, sem):
        wid = lax.axis_index("s") * info.num_cores + lax.axis_index("c")
        base = wid * b_per_w
        pltpu.sync_copy(idx_hbm.at[pl.ds(base, b_per_w)], idx_v)
        pltpu.async_copy(table_hbm.at[idx_v], rows_v, sem).wait()
        pltpu.sync_copy(rows_v, out_hbm.at[pl.ds(base, b_per_w)])

    return sc_gather


@jax.jit
def kernel(z, codebook):
    z_flat = z.reshape(-1, EMBEDDING_DIM)
    n_rows = z_flat.shape[0]
    s1 = jnp.sum(z_flat ** 2, axis=1, keepdims=True)
    s2 = jnp.sum(codebook ** 2, axis=1).reshape(1, -1)
    cbm2 = -2.0 * codebook

    grid = (n_rows // M_BLK,)
    idx = pl.pallas_call(
        _argmin_kernel,
        grid=grid,
        in_specs=[
            pl.BlockSpec((M_BLK, EMBEDDING_DIM), lambda i: (i, 0)),
            pl.BlockSpec((N_EMBEDDINGS, EMBEDDING_DIM), lambda i: (0, 0)),
            pl.BlockSpec((M_BLK, 1), lambda i: (i, 0)),
            pl.BlockSpec((1, N_EMBEDDINGS), lambda i: (0, 0)),
        ],
        out_specs=pl.BlockSpec((M_BLK, 1), lambda i: (i, 0)),
        out_shape=jax.ShapeDtypeStruct((n_rows, 1), jnp.int32),
    )(z_flat, cbm2, s1, s2)

    table128 = codebook.reshape(N_EMBEDDINGS // 2, GATHER_W)
    idx_flat = idx.reshape(-1)
    z_q = _make_sc_gather(n_rows)(table128, idx_flat >> 1)

    t, loss_sum = pl.pallas_call(
        _epilogue_kernel,
        grid=grid,
        in_specs=[
            pl.BlockSpec((M_BLK, EMBEDDING_DIM), lambda i: (i, 0)),
            pl.BlockSpec((M_BLK, GATHER_W), lambda i: (i, 0)),
            pl.BlockSpec((M_BLK, 1), lambda i: (i, 0)),
        ],
        out_specs=[
            pl.BlockSpec((M_BLK, EMBEDDING_DIM), lambda i: (i, 0)),
            pl.BlockSpec((1, 1), lambda i: (0, 0)),
        ],
        out_shape=[
            jax.ShapeDtypeStruct((n_rows, EMBEDDING_DIM), jnp.float32),
            jax.ShapeDtypeStruct((1, 1), jnp.float32),
        ],
    )(z_flat, z_q, idx)

    mean_sq = loss_sum[0, 0] / (n_rows * EMBEDDING_DIM)
    embedding_loss = mean_sq + BETA * mean_sq
    z_q_out = z + t.reshape(z.shape)
    return z_q_out, embedding_loss
